# baseline (speedup 1.0000x reference)
import functools

import jax
import jax.numpy as jnp
from jax import lax
from jax.experimental import pallas as pl
from jax.experimental.pallas import tpu as pltpu
from jax.experimental.pallas import tpu_sc as plsc

NC = 2
NS = 16
L = 16
NW = NC * NS
CH = 128
DCH = 128
COL_BITS = 14
EPS = 1e-5


def _cdiv(a, b):
    return -(-a // b)


def _pick_block(n, cap=1024):
    for bn in range(min(n, cap), 0, -1):
        if n % bn == 0 and (bn % 8 == 0 or bn == n):
            return bn
    return n


def _z_body(wrel_ref, x_ref, wb_ref, z_ref):
    r = pl.program_id(0)
    w = wrel_ref[r, 0] * wb_ref[0]
    for b in range(1, wb_ref.shape[0]):
        w += wrel_ref[r, b] * wb_ref[b]
    z_ref[0] = jnp.dot(x_ref[...], w, preferred_element_type=jnp.float32,
                       precision=lax.Precision.HIGHEST)


def _compute_z(X, w_bases, w_rel):
    n, d_in = X.shape
    r, b = w_rel.shape
    d_out = w_bases.shape[2]
    bn = _pick_block(n)
    return pl.pallas_call(
        _z_body,
        grid=(r, n // bn),
        in_specs=[
            pl.BlockSpec(memory_space=pltpu.SMEM),
            pl.BlockSpec((bn, d_in), lambda i, j: (j, 0)),
            pl.BlockSpec((b, d_in, d_out), lambda i, j: (0, 0, 0)),
        ],
        out_specs=pl.BlockSpec((1, bn, d_out), lambda i, j: (i, j, 0)),
        out_shape=jax.ShapeDtypeStruct((r, n, d_out), jnp.float32),
    )(w_rel, X, w_bases)


def _add_body(a_ref, b_ref, o_ref):
    o_ref[...] = a_ref[...] + b_ref[...]


def _combine(p0, p1):
    n, d = p0.shape
    bn = _pick_block(n)
    return pl.pallas_call(
        _add_body,
        grid=(n // bn,),
        in_specs=[pl.BlockSpec((bn, d), lambda i: (i, 0)),
                  pl.BlockSpec((bn, d), lambda i: (i, 0))],
        out_specs=pl.BlockSpec((bn, d), lambda i: (i, 0)),
        out_shape=jax.ShapeDtypeStruct((n, d), jnp.float32),
    )(p0, p1)


def _make_deg_kernel(nch1, deg_pad):
    mesh = plsc.VectorSubcoreMesh(core_axis_name="c", subcore_axis_name="s")
    dsl = deg_pad // NS

    @functools.partial(
        pl.kernel, mesh=mesh,
        out_type=jax.ShapeDtypeStruct((deg_pad,), jnp.float32),
        scratch_types=[
            pltpu.VMEM((nch1 * DCH,), jnp.int32),
            pltpu.VMEM((DCH,), jnp.int32),
            pltpu.VMEM((DCH,), jnp.float32),
            pltpu.VMEM((dsl,), jnp.float32),
            pltpu.VMEM_SHARED((deg_pad,), jnp.float32),
        ])
    def deg_kernel(meta_hbm, degr_hbm, meta_t, didx_v, ones_v, wb_v, deg_sp):
        cid = lax.axis_index("c")
        sid = lax.axis_index("s")

        @pl.when(cid == 0)
        def _():
            zero = jnp.zeros((L,), jnp.float32)

            def zb(i, c):
                wb_v[pl.ds(i * L, L)] = zero
                return c
            lax.fori_loop(0, dsl // L, zb, None)
            d0 = sid * dsl
            pltpu.sync_copy(wb_v, deg_sp.at[pl.ds(d0, dsl)])
            one = jnp.ones((L,), jnp.float32)
            for j in range(DCH // L):
                ones_v[pl.ds(j * L, L)] = one
            pltpu.sync_copy(meta_hbm.at[pl.ds(sid * nch1 * DCH, nch1 * DCH)],
                            meta_t)
            plsc.subcore_barrier()

            def body(k, c):
                for j in range(DCH // L):
                    m16 = meta_t[pl.ds(k * DCH + j * L, L)]
                    didx_v[pl.ds(j * L, L)] = (
                        lax.shift_right_logical(m16, COL_BITS))
                pltpu.sync_copy(ones_v, deg_sp.at[didx_v], add=True)
                return c
            lax.fori_loop(0, nch1, body, None)
            plsc.subcore_barrier()

            pltpu.sync_copy(deg_sp.at[pl.ds(d0, dsl)], wb_v)

            def rb(i, c):
                sl = pl.ds(i * L, L)
                wb_v[sl] = 1.0 / (wb_v[sl] + EPS)
                return c
            lax.fori_loop(0, dsl // L, rb, None)
            pltpu.sync_copy(wb_v, degr_hbm.at[pl.ds(d0, dsl)])

    return deg_kernel


def _make_main_kernel(nch, n_pad, r_rel, n_nodes, d):
    mesh = plsc.VectorSubcoreMesh(core_axis_name="c", subcore_axis_name="s")
    rpt = n_pad // NS

    @functools.partial(
        pl.kernel, mesh=mesh,
        out_type=(jax.ShapeDtypeStruct((n_pad, d), jnp.float32),
                  jax.ShapeDtypeStruct((n_pad, d), jnp.float32)),
        scratch_types=[
            pltpu.VMEM((nch * CH,), jnp.int32),
            pltpu.VMEM((CH,), jnp.int32),
            pltpu.VMEM((CH,), jnp.int32),
            pltpu.VMEM((CH,), jnp.int32),
            pltpu.VMEM((CH + L,), jnp.float32),
            pltpu.VMEM((CH, d), jnp.float32),
            pltpu.VMEM_SHARED((n_pad, d), jnp.float32),
            pltpu.SemaphoreType.DMA,
        ])
    def main_kernel(meta_hbm, z_hbm, degr_hbm, p0_hbm, p1_hbm,
                    meta_t, row_v, didx_v, zidx_v, c_v, zbuf, acc, sem):
        cid = lax.axis_index("c")
        sid = lax.axis_index("s")
        wid = cid * NS + sid
        tb = max((r_rel - 1).bit_length(), 1)

        zero = jnp.zeros((L,), jnp.float32)

        def zb(i, c):
            for j in range(d // L):
                zbuf[i, pl.ds(j * L, L)] = zero
            return c
        lax.fori_loop(0, CH, zb, None)
        base = sid * rpt
        off = 0
        while off < rpt:
            cnt = min(CH, rpt - off)
            pltpu.sync_copy(zbuf.at[pl.ds(0, cnt)],
                            acc.at[pl.ds(base + off, cnt)])
            off += cnt

        pltpu.sync_copy(meta_hbm.at[pl.ds(wid * nch * CH, nch * CH)], meta_t)
        plsc.subcore_barrier()

        def body(k, carry):
            for j in range(CH // L):
                sl = pl.ds(j * L, L)
                m16 = meta_t[pl.ds(k * CH + j * L, L)]
                rt16 = lax.shift_right_logical(m16, COL_BITS)
                row16 = lax.shift_right_logical(rt16, tb)
                t16 = rt16 - lax.shift_left(row16, tb)
                col16 = m16 - lax.shift_left(rt16, COL_BITS)
                row_v[sl] = row16
                didx_v[sl] = rt16
                zidx_v[sl] = t16 * n_nodes + col16
            pltpu.sync_copy(degr_hbm.at[didx_v], c_v.at[pl.ds(0, CH)])
            pltpu.async_copy(z_hbm.at[zidx_v], zbuf, sem).wait()

            def srow(i, c2):
                cs = c_v[pl.ds(i, L)][0]
                for j in range(d // L):
                    sl2 = pl.ds(j * L, L)
                    zbuf[i, sl2] = zbuf[i, sl2] * cs
                return c2
            lax.fori_loop(0, CH, srow, None)
            pltpu.sync_copy(zbuf, acc.at[row_v], add=True)
            return carry
        lax.fori_loop(0, nch, body, None)
        plsc.subcore_barrier()

        @pl.when(cid == 0)
        def _():
            off = 0
            while off < rpt:
                cnt = min(CH, rpt - off)
                pltpu.sync_copy(acc.at[pl.ds(base + off, cnt)],
                                p0_hbm.at[pl.ds(base + off, cnt)])
                off += cnt

        @pl.when(cid == 1)
        def _():
            off = 0
            while off < rpt:
                cnt = min(CH, rpt - off)
                pltpu.sync_copy(acc.at[pl.ds(base + off, cnt)],
                                p1_hbm.at[pl.ds(base + off, cnt)])
                off += cnt

    return main_kernel


def kernel(X, edge_index, edge_type, l, w_bases, w_rel):
    del l
    n, _ = X.shape
    r_rel, _ = w_rel.shape
    d_out = w_bases.shape[2]
    e = edge_type.shape[0]
    assert n < (1 << COL_BITS)

    tb = max((r_rel - 1).bit_length(), 1)
    unit = max(NW * CH * 8, NS * DCH * 8)
    assert unit % (NS * DCH * 8) == 0
    e_pad = _cdiv(e, unit) * unit
    pad = e_pad - e
    row = jnp.concatenate([edge_index[0], jnp.full((pad,), n, jnp.int32)])
    col = jnp.concatenate([edge_index[1], jnp.zeros((pad,), jnp.int32)])
    typ = jnp.concatenate([edge_type, jnp.zeros((pad,), jnp.int32)])
    meta = (((row << tb) | typ) << COL_BITS) | col

    z = _compute_z(X, w_bases, w_rel)
    z2 = z.reshape(r_rel * n, d_out)

    deg_pad = _cdiv((n + 1) << tb, NS * L) * NS * L
    n_pad = _cdiv(n + 1, NS * 8) * NS * 8
    degr = _make_deg_kernel(e_pad // (NS * DCH), deg_pad)(meta)
    p0, p1 = _make_main_kernel(e_pad // (NW * CH), n_pad, r_rel, n, d_out)(
        meta, z2, degr)
    return _combine(p0[:n], p1[:n])


# speedup vs baseline: 1.3313x; 1.3313x over previous
import functools

import jax
import jax.numpy as jnp
from jax import lax
from jax.experimental import pallas as pl
from jax.experimental.pallas import tpu as pltpu
from jax.experimental.pallas import tpu_sc as plsc

NC = 2
NS = 16
L = 16
NW = NC * NS
CH = 128
DCH = 128
COL_BITS = 14
EPS = 1e-5


def _cdiv(a, b):
    return -(-a // b)


def _pick_block(n, cap=1024):
    for bn in range(min(n, cap), 0, -1):
        if n % bn == 0 and (bn % 8 == 0 or bn == n):
            return bn
    return n


def _z_body(wrel_ref, x_ref, wb_ref, z_ref):
    r = pl.program_id(0)
    w = wrel_ref[r, 0] * wb_ref[0]
    for b in range(1, wb_ref.shape[0]):
        w += wrel_ref[r, b] * wb_ref[b]
    z_ref[0] = jnp.dot(x_ref[...], w, preferred_element_type=jnp.float32,
                       precision=lax.Precision.HIGHEST)


def _compute_z(X, w_bases, w_rel):
    n, d_in = X.shape
    r, b = w_rel.shape
    d_out = w_bases.shape[2]
    bn = _pick_block(n)
    return pl.pallas_call(
        _z_body,
        grid=(r, n // bn),
        in_specs=[
            pl.BlockSpec(memory_space=pltpu.SMEM),
            pl.BlockSpec((bn, d_in), lambda i, j: (j, 0)),
            pl.BlockSpec((b, d_in, d_out), lambda i, j: (0, 0, 0)),
        ],
        out_specs=pl.BlockSpec((1, bn, d_out), lambda i, j: (i, j, 0)),
        out_shape=jax.ShapeDtypeStruct((r, n, d_out), jnp.float32),
    )(w_rel, X, w_bases)


def _add_body(a_ref, b_ref, o_ref):
    o_ref[...] = a_ref[...] + b_ref[...]


def _combine(p0, p1):
    n, d = p0.shape
    bn = _pick_block(n)
    return pl.pallas_call(
        _add_body,
        grid=(n // bn,),
        in_specs=[pl.BlockSpec((bn, d), lambda i: (i, 0)),
                  pl.BlockSpec((bn, d), lambda i: (i, 0))],
        out_specs=pl.BlockSpec((bn, d), lambda i: (i, 0)),
        out_shape=jax.ShapeDtypeStruct((n, d), jnp.float32),
    )(p0, p1)


def _make_deg_kernel(nch1, deg_pad):
    mesh = plsc.VectorSubcoreMesh(core_axis_name="c", subcore_axis_name="s")
    dsl = deg_pad // NS

    @functools.partial(
        pl.kernel, mesh=mesh,
        out_type=jax.ShapeDtypeStruct((deg_pad,), jnp.float32),
        scratch_types=[
            pltpu.VMEM((nch1 * DCH,), jnp.int32),
            pltpu.VMEM((DCH,), jnp.int32),
            pltpu.VMEM((DCH,), jnp.float32),
            pltpu.VMEM((dsl,), jnp.float32),
            pltpu.VMEM_SHARED((deg_pad,), jnp.float32),
        ])
    def deg_kernel(meta_hbm, degr_hbm, meta_t, didx_v, ones_v, wb_v, deg_sp):
        cid = lax.axis_index("c")
        sid = lax.axis_index("s")

        @pl.when(cid == 0)
        def _():
            zero = jnp.zeros((L,), jnp.float32)

            def zb(i, c):
                wb_v[pl.ds(i * L, L)] = zero
                return c
            lax.fori_loop(0, dsl // L, zb, None)
            d0 = sid * dsl
            pltpu.sync_copy(wb_v, deg_sp.at[pl.ds(d0, dsl)])
            one = jnp.ones((L,), jnp.float32)
            for j in range(DCH // L):
                ones_v[pl.ds(j * L, L)] = one
            pltpu.sync_copy(meta_hbm.at[pl.ds(sid * nch1 * DCH, nch1 * DCH)],
                            meta_t)
            plsc.subcore_barrier()

            def body(k, c):
                for j in range(DCH // L):
                    m16 = meta_t[pl.ds(k * DCH + j * L, L)]
                    didx_v[pl.ds(j * L, L)] = (
                        lax.shift_right_logical(m16, COL_BITS))
                pltpu.sync_copy(ones_v, deg_sp.at[didx_v], add=True)
                return c
            lax.fori_loop(0, nch1, body, None)
            plsc.subcore_barrier()

            pltpu.sync_copy(deg_sp.at[pl.ds(d0, dsl)], wb_v)

            def rb(i, c):
                sl = pl.ds(i * L, L)
                wb_v[sl] = 1.0 / (wb_v[sl] + EPS)
                return c
            lax.fori_loop(0, dsl // L, rb, None)
            pltpu.sync_copy(wb_v, degr_hbm.at[pl.ds(d0, dsl)])

    return deg_kernel


def _make_main_kernel(nch, n_pad, r_rel, n_nodes, d):
    mesh = plsc.VectorSubcoreMesh(core_axis_name="c", subcore_axis_name="s")
    rpt = n_pad // NS

    @functools.partial(
        pl.kernel, mesh=mesh,
        out_type=(jax.ShapeDtypeStruct((n_pad, d), jnp.float32),
                  jax.ShapeDtypeStruct((n_pad, d), jnp.float32)),
        scratch_types=[
            pltpu.VMEM((2, CH), jnp.int32),
            pltpu.VMEM((CH,), jnp.int32),
            pltpu.VMEM((CH,), jnp.int32),
            pltpu.VMEM((CH,), jnp.int32),
            pltpu.VMEM((CH,), jnp.int32),
            pltpu.VMEM((CH,), jnp.int32),
            pltpu.VMEM((CH,), jnp.int32),
            pltpu.VMEM((CH + L,), jnp.float32),
            pltpu.VMEM((CH + L,), jnp.float32),
            pltpu.VMEM((2, CH, d), jnp.float32),
            pltpu.VMEM_SHARED((n_pad, d), jnp.float32),
            pltpu.SemaphoreType.DMA,
            pltpu.SemaphoreType.DMA,
            pltpu.SemaphoreType.DMA,
            pltpu.SemaphoreType.DMA,
            pltpu.SemaphoreType.DMA,
            pltpu.SemaphoreType.DMA,
            pltpu.SemaphoreType.DMA,
            pltpu.SemaphoreType.DMA,
        ])
    def main_kernel(meta_hbm, z_hbm, degr_hbm, p0_hbm, p1_hbm,
                    meta2, row_v0, row_v1, didx_v0, didx_v1, zidx_v0,
                    zidx_v1, c_v0, c_v1, zbuf2, acc,
                    msem0, msem1, zsem0, zsem1, dsem0, dsem1, ssem0, ssem1):
        cid = lax.axis_index("c")
        sid = lax.axis_index("s")
        wid = cid * NS + sid
        tb = max((r_rel - 1).bit_length(), 1)
        base_m = wid * (nch * CH)
        row_v = (row_v0, row_v1)
        didx_v = (didx_v0, didx_v1)
        zidx_v = (zidx_v0, zidx_v1)
        c_v = (c_v0, c_v1)
        msem = (msem0, msem1)
        zsem = (zsem0, zsem1)
        dsem = (dsem0, dsem1)
        ssem = (ssem0, ssem1)

        def fire_meta(k2, s):
            pltpu.async_copy(meta_hbm.at[pl.ds(base_m + k2 * CH, CH)],
                             meta2.at[s], msem[s])

        def wait_meta(s):
            pltpu.make_async_copy(meta_hbm.at[pl.ds(base_m, CH)],
                                  meta2.at[s], msem[s]).wait()

        def compute_idx(s):
            for j in range(CH // L):
                sl = pl.ds(j * L, L)
                m16 = meta2[s, sl]
                rt16 = lax.shift_right_logical(m16, COL_BITS)
                row16 = lax.shift_right_logical(rt16, tb)
                t16 = rt16 - lax.shift_left(row16, tb)
                col16 = m16 - lax.shift_left(rt16, COL_BITS)
                row_v[s][sl] = row16
                didx_v[s][sl] = rt16
                zidx_v[s][sl] = t16 * n_nodes + col16

        def fire_gather(s):
            pltpu.async_copy(z_hbm.at[zidx_v[s]], zbuf2.at[s], zsem[s])
            pltpu.async_copy(degr_hbm.at[didx_v[s]],
                             c_v[s].at[pl.ds(0, CH)], dsem[s])

        def wait_gather(s):
            pltpu.make_async_copy(z_hbm.at[zidx_v[s]], zbuf2.at[s],
                                  zsem[s]).wait()
            pltpu.make_async_copy(degr_hbm.at[didx_v[s]],
                                  c_v[s].at[pl.ds(0, CH)], dsem[s]).wait()

        def fire_scatter(s):
            pltpu.async_copy(zbuf2.at[s], acc.at[row_v[s]], ssem[s],
                             add=True)

        def wait_scatter(s):
            pltpu.make_async_copy(zbuf2.at[s], acc.at[row_v[s]],
                                  ssem[s]).wait()

        def scale(s):
            def srow(i, c3):
                cs = c_v[s][pl.ds(i, L)][0]
                for j in range(d // L):
                    sl2 = pl.ds(j * L, L)
                    zbuf2[s, i, sl2] = zbuf2[s, i, sl2] * cs
                return c3
            lax.fori_loop(0, CH, srow, None)

        zero = jnp.zeros((L,), jnp.float32)

        def zb(i, c):
            for j in range(d // L):
                zbuf2[0, i, pl.ds(j * L, L)] = zero
            return c
        lax.fori_loop(0, CH, zb, None)
        base = sid * rpt
        off = 0
        while off < rpt:
            cnt = min(CH, rpt - off)
            pltpu.sync_copy(zbuf2.at[0, pl.ds(0, cnt)],
                            acc.at[pl.ds(base + off, cnt)])
            off += cnt
        plsc.subcore_barrier()

        fire_meta(0, 0)
        if nch > 1:
            fire_meta(1, 1)
        wait_meta(0)
        compute_idx(0)
        fire_gather(0)

        def body(k, carry):
            par = k % 2

            def steps(s):
                s1 = 1 - s

                @pl.when(k + 2 < nch)
                def _():
                    fire_meta(k + 2, s)

                @pl.when(k + 1 < nch)
                def _():
                    @pl.when(k >= 1)
                    def _():
                        wait_scatter(s1)
                    wait_meta(s1)
                    compute_idx(s1)
                    fire_gather(s1)

                wait_gather(s)
                scale(s)
                fire_scatter(s)

            @pl.when(par == 0)
            def _():
                steps(0)

            @pl.when(par == 1)
            def _():
                steps(1)
            return carry
        lax.fori_loop(0, nch, body, None)

        wait_scatter((nch - 1) % 2)
        if nch > 1:
            wait_scatter(nch % 2)
        plsc.subcore_barrier()

        @pl.when(cid == 0)
        def _():
            off = 0
            while off < rpt:
                cnt = min(CH, rpt - off)
                pltpu.sync_copy(acc.at[pl.ds(base + off, cnt)],
                                p0_hbm.at[pl.ds(base + off, cnt)])
                off += cnt

        @pl.when(cid == 1)
        def _():
            off = 0
            while off < rpt:
                cnt = min(CH, rpt - off)
                pltpu.sync_copy(acc.at[pl.ds(base + off, cnt)],
                                p1_hbm.at[pl.ds(base + off, cnt)])
                off += cnt

    return main_kernel


def kernel(X, edge_index, edge_type, l, w_bases, w_rel):
    del l
    n, _ = X.shape
    r_rel, _ = w_rel.shape
    d_out = w_bases.shape[2]
    e = edge_type.shape[0]
    assert n < (1 << COL_BITS)

    tb = max((r_rel - 1).bit_length(), 1)
    unit = max(NW * CH * 8, NS * DCH * 8)
    assert unit % (NS * DCH * 8) == 0
    e_pad = _cdiv(e, unit) * unit
    pad = e_pad - e
    row = jnp.concatenate([edge_index[0], jnp.full((pad,), n, jnp.int32)])
    col = jnp.concatenate([edge_index[1], jnp.zeros((pad,), jnp.int32)])
    typ = jnp.concatenate([edge_type, jnp.zeros((pad,), jnp.int32)])
    meta = (((row << tb) | typ) << COL_BITS) | col

    z = _compute_z(X, w_bases, w_rel)
    z2 = z.reshape(r_rel * n, d_out)

    deg_pad = _cdiv((n + 1) << tb, NS * L) * NS * L
    n_pad = _cdiv(n + 1, NS * 8) * NS * 8
    degr = _make_deg_kernel(e_pad // (NS * DCH), deg_pad)(meta)
    p0, p1 = _make_main_kernel(e_pad // (NW * CH), n_pad, r_rel, n, d_out)(
        meta, z2, degr)
    return _combine(p0[:n], p1[:n])


# speedup vs baseline: 1.3318x; 1.0004x over previous
import functools

import jax
import jax.numpy as jnp
from jax import lax
from jax.experimental import pallas as pl
from jax.experimental.pallas import tpu as pltpu
from jax.experimental.pallas import tpu_sc as plsc

NC = 2
NS = 16
L = 16
NW = NC * NS
CH = 128
DCH = 128
COL_BITS = 14
EPS = 1e-5


def _cdiv(a, b):
    return -(-a // b)


def _pick_block(n, cap=1024):
    for bn in range(min(n, cap), 0, -1):
        if n % bn == 0 and (bn % 8 == 0 or bn == n):
            return bn
    return n


def _z_body(wrel_ref, x_ref, wb_ref, z_ref):
    r = pl.program_id(0)
    w = wrel_ref[r, 0] * wb_ref[0]
    for b in range(1, wb_ref.shape[0]):
        w += wrel_ref[r, b] * wb_ref[b]
    z_ref[0] = jnp.dot(x_ref[...], w, preferred_element_type=jnp.float32,
                       precision=lax.Precision.HIGHEST)


def _compute_z(X, w_bases, w_rel):
    n, d_in = X.shape
    r, b = w_rel.shape
    d_out = w_bases.shape[2]
    bn = _pick_block(n)
    return pl.pallas_call(
        _z_body,
        grid=(r, n // bn),
        in_specs=[
            pl.BlockSpec(memory_space=pltpu.SMEM),
            pl.BlockSpec((bn, d_in), lambda i, j: (j, 0)),
            pl.BlockSpec((b, d_in, d_out), lambda i, j: (0, 0, 0)),
        ],
        out_specs=pl.BlockSpec((1, bn, d_out), lambda i, j: (i, j, 0)),
        out_shape=jax.ShapeDtypeStruct((r, n, d_out), jnp.float32),
    )(w_rel, X, w_bases)


def _add_body(a_ref, b_ref, o_ref):
    o_ref[...] = a_ref[...] + b_ref[...]


def _combine(p0, p1):
    n, d = p0.shape
    bn = _pick_block(n)
    return pl.pallas_call(
        _add_body,
        grid=(n // bn,),
        in_specs=[pl.BlockSpec((bn, d), lambda i: (i, 0)),
                  pl.BlockSpec((bn, d), lambda i: (i, 0))],
        out_specs=pl.BlockSpec((bn, d), lambda i: (i, 0)),
        out_shape=jax.ShapeDtypeStruct((n, d), jnp.float32),
    )(p0, p1)


def _make_deg_kernel(nch1, deg_pad):
    mesh = plsc.VectorSubcoreMesh(core_axis_name="c", subcore_axis_name="s")
    dsl = deg_pad // NS

    @functools.partial(
        pl.kernel, mesh=mesh,
        out_type=jax.ShapeDtypeStruct((deg_pad,), jnp.float32),
        scratch_types=[
            pltpu.VMEM((nch1 * DCH,), jnp.int32),
            pltpu.VMEM((DCH,), jnp.int32),
            pltpu.VMEM((DCH,), jnp.float32),
            pltpu.VMEM((dsl,), jnp.float32),
            pltpu.VMEM_SHARED((deg_pad,), jnp.float32),
        ])
    def deg_kernel(meta_hbm, degr_hbm, meta_t, didx_v, ones_v, wb_v, deg_sp):
        cid = lax.axis_index("c")
        sid = lax.axis_index("s")

        @pl.when(cid == 0)
        def _():
            zero = jnp.zeros((L,), jnp.float32)

            def zb(i, c):
                wb_v[pl.ds(i * L, L)] = zero
                return c
            lax.fori_loop(0, dsl // L, zb, None)
            d0 = sid * dsl
            pltpu.sync_copy(wb_v, deg_sp.at[pl.ds(d0, dsl)])
            one = jnp.ones((L,), jnp.float32)
            for j in range(DCH // L):
                ones_v[pl.ds(j * L, L)] = one
            pltpu.sync_copy(meta_hbm.at[pl.ds(sid * nch1 * DCH, nch1 * DCH)],
                            meta_t)
            plsc.subcore_barrier()

            def body(k, c):
                for j in range(DCH // L):
                    m16 = meta_t[pl.ds(k * DCH + j * L, L)]
                    didx_v[pl.ds(j * L, L)] = (
                        lax.shift_right_logical(m16, COL_BITS))
                pltpu.sync_copy(ones_v, deg_sp.at[didx_v], add=True)
                return c
            lax.fori_loop(0, nch1, body, None)
            plsc.subcore_barrier()

            pltpu.sync_copy(deg_sp.at[pl.ds(d0, dsl)], wb_v)

            def rb(i, c):
                sl = pl.ds(i * L, L)
                wb_v[sl] = 1.0 / (wb_v[sl] + EPS)
                return c
            lax.fori_loop(0, dsl // L, rb, None)
            pltpu.sync_copy(wb_v, degr_hbm.at[pl.ds(d0, dsl)])

    return deg_kernel


def _make_main_kernel(nch, n_pad, r_rel, n_nodes, d):
    mesh = plsc.VectorSubcoreMesh(core_axis_name="c", subcore_axis_name="s")
    rpt = n_pad // NS

    @functools.partial(
        pl.kernel, mesh=mesh,
        out_type=(jax.ShapeDtypeStruct((n_pad, d), jnp.float32),
                  jax.ShapeDtypeStruct((n_pad, d), jnp.float32)),
        scratch_types=[
            pltpu.VMEM((2, CH), jnp.int32),
            pltpu.VMEM((CH,), jnp.int32),
            pltpu.VMEM((CH,), jnp.int32),
            pltpu.VMEM((CH,), jnp.int32),
            pltpu.VMEM((CH,), jnp.int32),
            pltpu.VMEM((CH,), jnp.int32),
            pltpu.VMEM((CH,), jnp.int32),
            pltpu.VMEM((CH + L,), jnp.float32),
            pltpu.VMEM((CH + L,), jnp.float32),
            pltpu.VMEM((2, CH, d), jnp.float32),
            pltpu.VMEM_SHARED((n_pad, d), jnp.float32),
            pltpu.SemaphoreType.DMA,
            pltpu.SemaphoreType.DMA,
            pltpu.SemaphoreType.DMA,
            pltpu.SemaphoreType.DMA,
            pltpu.SemaphoreType.DMA,
            pltpu.SemaphoreType.DMA,
            pltpu.SemaphoreType.DMA,
            pltpu.SemaphoreType.DMA,
        ])
    def main_kernel(meta_hbm, z_hbm, degr_hbm, p0_hbm, p1_hbm,
                    meta2, row_v0, row_v1, didx_v0, didx_v1, zidx_v0,
                    zidx_v1, c_v0, c_v1, zbuf2, acc,
                    msem0, msem1, zsem0, zsem1, dsem0, dsem1, ssem0, ssem1):
        cid = lax.axis_index("c")
        sid = lax.axis_index("s")
        wid = cid * NS + sid
        tb = max((r_rel - 1).bit_length(), 1)
        base_m = wid * (nch * CH)
        row_v = (row_v0, row_v1)
        didx_v = (didx_v0, didx_v1)
        zidx_v = (zidx_v0, zidx_v1)
        c_v = (c_v0, c_v1)
        msem = (msem0, msem1)
        zsem = (zsem0, zsem1)
        dsem = (dsem0, dsem1)
        ssem = (ssem0, ssem1)

        def fire_meta(k2, s):
            pltpu.async_copy(meta_hbm.at[pl.ds(base_m + k2 * CH, CH)],
                             meta2.at[s], msem[s])

        def wait_meta(s):
            pltpu.make_async_copy(meta_hbm.at[pl.ds(base_m, CH)],
                                  meta2.at[s], msem[s]).wait()

        def compute_idx(s):
            for j in range(CH // L):
                sl = pl.ds(j * L, L)
                m16 = meta2[s, sl]
                rt16 = lax.shift_right_logical(m16, COL_BITS)
                row16 = lax.shift_right_logical(rt16, tb)
                t16 = rt16 - lax.shift_left(row16, tb)
                col16 = m16 - lax.shift_left(rt16, COL_BITS)
                row_v[s][sl] = row16
                didx_v[s][sl] = rt16
                zidx_v[s][sl] = t16 * n_nodes + col16

        def fire_gather(s):
            h = CH // 2
            pltpu.async_copy(z_hbm.at[zidx_v[s].at[pl.ds(0, h)]],
                             zbuf2.at[s, pl.ds(0, h)], zsem[s])
            pltpu.async_copy(z_hbm.at[zidx_v[s].at[pl.ds(h, h)]],
                             zbuf2.at[s, pl.ds(h, h)], zsem[s])
            pltpu.async_copy(degr_hbm.at[didx_v[s]],
                             c_v[s].at[pl.ds(0, CH)], dsem[s])

        def wait_gather(s):
            h = CH // 2
            pltpu.make_async_copy(z_hbm.at[zidx_v[s].at[pl.ds(0, h)]],
                                  zbuf2.at[s, pl.ds(0, h)], zsem[s]).wait()
            pltpu.make_async_copy(z_hbm.at[zidx_v[s].at[pl.ds(h, h)]],
                                  zbuf2.at[s, pl.ds(h, h)], zsem[s]).wait()
            pltpu.make_async_copy(degr_hbm.at[didx_v[s]],
                                  c_v[s].at[pl.ds(0, CH)], dsem[s]).wait()

        def fire_scatter(s):
            pltpu.async_copy(zbuf2.at[s], acc.at[row_v[s]], ssem[s],
                             add=True)

        def wait_scatter(s):
            pltpu.make_async_copy(zbuf2.at[s], acc.at[row_v[s]],
                                  ssem[s]).wait()

        def scale(s):
            def srow(i, c3):
                cs = c_v[s][pl.ds(i, L)][0]
                for j in range(d // L):
                    sl2 = pl.ds(j * L, L)
                    zbuf2[s, i, sl2] = zbuf2[s, i, sl2] * cs
                return c3
            lax.fori_loop(0, CH, srow, None)

        zero = jnp.zeros((L,), jnp.float32)

        def zb(i, c):
            for j in range(d // L):
                zbuf2[0, i, pl.ds(j * L, L)] = zero
            return c
        lax.fori_loop(0, CH, zb, None)
        base = sid * rpt
        off = 0
        while off < rpt:
            cnt = min(CH, rpt - off)
            pltpu.sync_copy(zbuf2.at[0, pl.ds(0, cnt)],
                            acc.at[pl.ds(base + off, cnt)])
            off += cnt
        plsc.subcore_barrier()

        fire_meta(0, 0)
        if nch > 1:
            fire_meta(1, 1)
        wait_meta(0)
        compute_idx(0)
        fire_gather(0)

        def body(k, carry):
            par = k % 2

            def steps(s):
                s1 = 1 - s

                @pl.when(k + 2 < nch)
                def _():
                    fire_meta(k + 2, s)

                @pl.when(k + 1 < nch)
                def _():
                    @pl.when(k >= 1)
                    def _():
                        wait_scatter(s1)
                    wait_meta(s1)
                    compute_idx(s1)
                    fire_gather(s1)

                wait_gather(s)
                scale(s)
                fire_scatter(s)

            @pl.when(par == 0)
            def _():
                steps(0)

            @pl.when(par == 1)
            def _():
                steps(1)
            return carry
        lax.fori_loop(0, nch, body, None)

        wait_scatter((nch - 1) % 2)
        if nch > 1:
            wait_scatter(nch % 2)
        plsc.subcore_barrier()

        @pl.when(cid == 0)
        def _():
            off = 0
            while off < rpt:
                cnt = min(CH, rpt - off)
                pltpu.sync_copy(acc.at[pl.ds(base + off, cnt)],
                                p0_hbm.at[pl.ds(base + off, cnt)])
                off += cnt

        @pl.when(cid == 1)
        def _():
            off = 0
            while off < rpt:
                cnt = min(CH, rpt - off)
                pltpu.sync_copy(acc.at[pl.ds(base + off, cnt)],
                                p1_hbm.at[pl.ds(base + off, cnt)])
                off += cnt

    return main_kernel


def kernel(X, edge_index, edge_type, l, w_bases, w_rel):
    del l
    n, _ = X.shape
    r_rel, _ = w_rel.shape
    d_out = w_bases.shape[2]
    e = edge_type.shape[0]
    assert n < (1 << COL_BITS)

    tb = max((r_rel - 1).bit_length(), 1)
    unit = max(NW * CH * 8, NS * DCH * 8)
    assert unit % (NS * DCH * 8) == 0
    e_pad = _cdiv(e, unit) * unit
    pad = e_pad - e
    row = jnp.concatenate([edge_index[0], jnp.full((pad,), n, jnp.int32)])
    col = jnp.concatenate([edge_index[1], jnp.zeros((pad,), jnp.int32)])
    typ = jnp.concatenate([edge_type, jnp.zeros((pad,), jnp.int32)])
    meta = (((row << tb) | typ) << COL_BITS) | col

    z = _compute_z(X, w_bases, w_rel)
    z2 = z.reshape(r_rel * n, d_out)

    deg_pad = _cdiv((n + 1) << tb, NS * L) * NS * L
    n_pad = _cdiv(n + 1, NS * 8) * NS * 8
    degr = _make_deg_kernel(e_pad // (NS * DCH), deg_pad)(meta)
    p0, p1 = _make_main_kernel(e_pad // (NW * CH), n_pad, r_rel, n, d_out)(
        meta, z2, degr)
    return _combine(p0[:n], p1[:n])
